# baseline (speedup 1.0000x reference)
import functools

import jax
import jax.numpy as jnp
import numpy as np
from jax import lax
from jax.experimental import pallas as pl
from jax.experimental.pallas import tpu as pltpu
from jax.experimental.pallas import tpu_sc as plsc

B = 16384
V = 1000000
D = 64
P = 64

VT = 999936
NT = VT // 128
GW = NT * 512
HW = D // 2

_NC = 2
_NS = 16
_NW = _NC * _NS
_B_PER_W = B // _NW
_CHUNK = 128
_NCHUNK = _B_PER_W // _CHUNK

_PERM = np.concatenate([np.arange(0, D, 2), np.arange(1, D, 2)])


@functools.cache
def _make_sc_gather():
    mesh = plsc.VectorSubcoreMesh(core_axis_name="c", subcore_axis_name="s")

    @functools.partial(
        pl.kernel,
        mesh=mesh,
        out_type=jax.ShapeDtypeStruct((B * HW,), jnp.float32),
        scratch_types=[
            pltpu.VMEM((_B_PER_W,), jnp.int32),
            pltpu.VMEM((4, _CHUNK), jnp.int32),
            pltpu.VMEM((HW * _CHUNK,), jnp.float32),
            pltpu.SemaphoreType.DMA,
        ],
        compiler_params=pltpu.CompilerParams(use_tc_tiling_on_sc=False),
    )
    def gather_kernel(g0, g1, g2, g3, g4, g5, g6, g7, ids_hbm,
                      out_hbm, ids_v, widx_v, stage_v, sem):
        groups = (g0, g1, g2, g3, g4, g5, g6, g7)
        wid = lax.axis_index("s") * _NC + lax.axis_index("c")
        base = wid * _B_PER_W
        pltpu.sync_copy(ids_hbm.at[pl.ds(base, _B_PER_W)], ids_v)
        for c in range(_NCHUNK):
            for s in range(8):
                v = ids_v[pl.ds(c * _CHUNK + 16 * s, 16)]
                vc = jnp.where(v >= VT, 0, v)
                wbase = (vc >> 7) * 512 + (vc & 127)
                for k in range(4):
                    widx_v[k, pl.ds(16 * s, 16)] = wbase + 128 * k
            copies = []
            for g in range(8):
                for k in range(4):
                    copies.append(pltpu.async_copy(
                        groups[g].at[widx_v.at[k]],
                        stage_v.at[pl.ds((4 * g + k) * _CHUNK, _CHUNK)],
                        sem))
            for cp in copies:
                cp.wait()
            pltpu.sync_copy(
                stage_v,
                out_hbm.at[pl.ds((base + c * _CHUNK) * HW, _CHUNK * HW)])

    return gather_kernel


def _dense_body(uep_ref, oh_ref, remp_ref, pft_ref, w1t_ref, b1_ref,
                w2t_ref, b2_ref, wfup_ref, wfpt_ref, bf_ref, out_ref):
    ui = lax.bitcast_convert_type(uep_ref[...], jnp.int32)
    lo = lax.bitcast_convert_type(ui << 16, jnp.float32)
    hi = lax.bitcast_convert_type(ui & jnp.int32(-65536), jnp.float32)
    uep = jnp.concatenate([lo, hi], axis=0)
    oh = oh_ref[...]
    m = jnp.sum(oh, axis=0, keepdims=True)
    uep = (uep * (1.0 - m)
           + jnp.dot(remp_ref[...], oh, preferred_element_type=jnp.float32))
    ht = jnp.maximum(
        jnp.dot(w1t_ref[...], pft_ref[...],
                preferred_element_type=jnp.float32) + b1_ref[...], 0.0)
    pet = (jnp.dot(w2t_ref[...], ht, preferred_element_type=jnp.float32)
           + b2_ref[...])
    acc = (jnp.dot(wfup_ref[...], uep, preferred_element_type=jnp.float32)
           + jnp.dot(wfpt_ref[...], pet, preferred_element_type=jnp.float32)
           + bf_ref[...])
    out_ref[...] = jnp.tanh(acc)


_BN = 2048


def _dense(uep, oh, remp, pft, W1t, b1c, W2t, b2c, Wfup, Wfpt, bfc):
    grid = (B // _BN,)

    def full(r, c):
        return pl.BlockSpec((r, c), lambda i: (0, 0))

    return pl.pallas_call(
        _dense_body,
        grid=grid,
        in_specs=[
            pl.BlockSpec((HW, _BN), lambda i: (0, i)),
            pl.BlockSpec((D, _BN), lambda i: (0, i)),
            full(D, D),
            pl.BlockSpec((P, _BN), lambda i: (0, i)),
            full(D // 2, P),
            full(D // 2, 1),
            full(D, D // 2),
            full(D, 1),
            full(D, D),
            full(D, D),
            full(D, 1),
        ],
        out_specs=pl.BlockSpec((D, _BN), lambda i: (0, i)),
        out_shape=jax.ShapeDtypeStruct((D, B), jnp.float32),
    )(uep, oh, remp, pft, W1t, b1c, W2t, b2c, Wfup, Wfpt, bfc)


def kernel(user_ids, profile_features, table, W1, b1, W2, b2, Wf, bf):
    ids = user_ids.astype(jnp.int32)

    groups = []
    for g in range(8):
        grp = lax.slice(table, (0, 8 * g), (VT, 8 * g + 8))
        g3 = grp.T.reshape(8, NT, 128).transpose(1, 0, 2)
        e16 = lax.bitcast_convert_type(
            g3[:, 0::2, :].astype(jnp.bfloat16), jnp.uint16).astype(jnp.uint32)
        o16 = lax.bitcast_convert_type(
            g3[:, 1::2, :].astype(jnp.bfloat16), jnp.uint16).astype(jnp.uint32)
        packed = lax.bitcast_convert_type(e16 | (o16 << 16), jnp.float32)
        groups.append(packed.reshape(GW))

    out_flat = _make_sc_gather()(*groups, ids)
    uep = out_flat.reshape(B // _CHUNK, HW, _CHUNK).transpose(1, 0, 2)
    uep = uep.reshape(HW, B)

    oh = (jnp.arange(D, dtype=jnp.int32)[:, None]
          == (ids - VT)[None, :]).astype(jnp.float32)
    remp = table[VT:, :].T[_PERM]

    pft = profile_features.T
    out_t = _dense(
        uep, oh, remp, pft,
        W1.T, b1.reshape(-1, 1),
        W2.T, b2.reshape(-1, 1),
        Wf[:D].T[:, _PERM], Wf[D:].T, bf.reshape(-1, 1),
    )
    return out_t.T
